# baseline (speedup 1.0000x reference)
import functools

import jax
import jax.numpy as jnp
from jax import lax
from jax.experimental import pallas as pl
from jax.experimental.pallas import tpu as pltpu
from jax.experimental.pallas import tpu_sc as plsc

K = 4


def _tc_layer_body(n, cin, o, op, first, x_ref, w_ref, idx_ref, t_ref,
                   ctr_ref):
    b = pl.program_id(0)
    w = w_ref[...]
    if first:
        xc = x_ref[0]
        g = lax.dot_general(xc, xc, (((0,), (0,)), ((), ())),
                            preferred_element_type=jnp.float32)
        sqc = lax.dot_general(xc * xc, jnp.ones((cin, 1), jnp.float32),
                              (((0,), (0,)), ((), ())),
                              preferred_element_type=jnp.float32)
        tn = lax.dot_general(xc, w[:, :cin], (((0,), (1,)), ((), ())),
                             preferred_element_type=jnp.float32)
        tc = lax.dot_general(xc, w[:, cin:], (((0,), (1,)), ((), ())),
                             preferred_element_type=jnp.float32)
    else:
        xb = x_ref[...][:, :cin]
        sqc = jnp.sum(xb * xb, axis=1, keepdims=True)
        g = lax.dot_general(xb, xb, (((1,), (1,)), ((), ())),
                            preferred_element_type=jnp.float32)
        tn = lax.dot_general(xb, w[:, :cin], (((1,), (1,)), ((), ())),
                             preferred_element_type=jnp.float32)
        tc = lax.dot_general(xb, w[:, cin:], (((1,), (1,)), ((), ())),
                             preferred_element_type=jnp.float32)
    iota0 = lax.broadcasted_iota(jnp.int32, (n, n), 0)
    iotaf = iota0.astype(jnp.float32)
    diag = iota0 == lax.broadcasted_iota(jnp.int32, (n, n), 1)
    d = jnp.where(diag, -jnp.inf, 2.0 * g - sqc)
    idx_ref[0:1, :] = (lax.broadcasted_iota(jnp.int32, (1, n), 1) + b * n)
    for k in range(1, K):
        m = jnp.max(d, axis=0, keepdims=True)
        cand = jnp.where(d == m, iotaf, jnp.float32(n))
        ikf = jnp.min(cand, axis=0, keepdims=True)
        idx_ref[k:k + 1, :] = ikf.astype(jnp.int32) + b * n
        if k < K - 1:
            d = jnp.where(iotaf == ikf, -jnp.inf, d)
    idx_ref[K:, :] = jnp.zeros((8 - K, n), jnp.int32)
    t_ref[:, :o] = tn
    ctr_ref[:, :o] = tc
    if op > o:
        zpad = jnp.zeros((n, op - o), jnp.float32)
        t_ref[:, o:] = zpad
        ctr_ref[:, o:] = zpad


def _tc_layer(xp, w, h0, bsz, n, cin, op):
    o = w.shape[0]
    first = xp.ndim == 3
    if first:
        xspec = pl.BlockSpec((1, cin, n), lambda b, h0=h0: (b + h0, 0, 0))
    else:
        cp = xp.shape[1]
        xspec = pl.BlockSpec((n, cp), lambda b, h0=h0: (b + h0, 0))
    return pl.pallas_call(
        functools.partial(_tc_layer_body, n, cin, o, op, first),
        grid=(bsz,),
        in_specs=[
            xspec,
            pl.BlockSpec(w.shape, lambda b: (0, 0)),
        ],
        out_specs=[
            pl.BlockSpec((8, n), lambda b: (b, 0)),
            pl.BlockSpec((n, op), lambda b: (b, 0)),
            pl.BlockSpec((n, op), lambda b: (b, 0)),
        ],
        out_shape=[
            jax.ShapeDtypeStruct((bsz * 8, n), jnp.int32),
            jax.ShapeDtypeStruct((bsz * n, op), jnp.float32),
            jax.ShapeDtypeStruct((bsz * n, op), jnp.float32),
        ],
    )(xp, w)


def _sc_layer(t, ctr, idx):
    bn, o = t.shape
    n = idx.shape[1]
    nw = 32
    rows_w = bn // nw
    ch = 32
    nch = rows_w // ch
    mesh = plsc.VectorSubcoreMesh(core_axis_name="c", subcore_axis_name="s")

    @functools.partial(
        pl.kernel,
        mesh=mesh,
        out_type=jax.ShapeDtypeStruct((bn, o), jnp.float32),
        scratch_types=[
            pltpu.VMEM((8, rows_w), jnp.int32),
            pltpu.VMEM((ch, o), jnp.float32),
            pltpu.VMEM((ch, o), jnp.float32),
            pltpu.VMEM((ch, o), jnp.float32),
            pltpu.VMEM((ch, o), jnp.float32),
            pltpu.VMEM((ch, o), jnp.float32),
            pltpu.VMEM((ch, o), jnp.float32),
            pltpu.SemaphoreType.DMA,
        ],
    )
    def sc_k(t_hbm, ctr_hbm, idx_hbm, out_hbm, idx_v, r0v, r1v, r2v, r3v,
             ctr_v, out_v, sem):
        wid = lax.axis_index("s") * 2 + lax.axis_index("c")
        base = wid * rows_w
        bi = base // n
        col = base - bi * n
        pltpu.sync_copy(idx_hbm.at[pl.ds(bi * 8, 8), pl.ds(col, rows_w)],
                        idx_v)
        for ci in range(nch):
            p0 = base + ci * ch
            rbufs = (r0v, r1v, r2v, r3v)
            copies = [pltpu.async_copy(t_hbm.at[pl.ds(p0, ch)], r0v, sem)]
            copies += [
                pltpu.async_copy(
                    t_hbm.at[idx_v.at[k, pl.ds(ci * ch, ch)]], rbufs[k], sem)
                for k in range(1, K)]
            pltpu.sync_copy(ctr_hbm.at[pl.ds(p0, ch)], ctr_v)
            for cpy in copies:
                cpy.wait()

            def body(j, carry):
                s = pl.ds(j * 16, 16)
                for p in range(ch):
                    v = jnp.maximum(
                        jnp.maximum(r0v[p, s], r1v[p, s]),
                        jnp.maximum(r2v[p, s], r3v[p, s]))
                    out_v[p, s] = jnp.maximum(v + ctr_v[p, s], 0.0)
                return carry

            lax.fori_loop(0, o // 16, body, 0)
            pltpu.sync_copy(out_v, out_hbm.at[pl.ds(p0, ch)])

    return sc_k(t, ctr, idx)


def _tc_final_body(x1_ref, x2_ref, x3_ref, x4_ref, w5_ref, out_ref):
    cat = jnp.concatenate(
        [x1_ref[...][:, :64], x2_ref[...][:, :64], x3_ref[...], x4_ref[...]],
        axis=1)
    r = lax.dot_general(w5_ref[...], cat, (((1,), (1,)), ((), ())),
                        preferred_element_type=jnp.float32)
    out_ref[...] = jnp.tanh(r)[None]


def _tc_final(feats, w5, bsz, n):
    x1, x2, x3, x4 = feats
    specs = [pl.BlockSpec((n, f.shape[1]), lambda b: (b, 0)) for f in feats]
    return pl.pallas_call(
        _tc_final_body,
        grid=(bsz,),
        in_specs=specs + [pl.BlockSpec((512, 512), lambda b: (0, 0))],
        out_specs=pl.BlockSpec((1, 512, n), lambda b: (b, 0, 0)),
        out_shape=jax.ShapeDtypeStruct((bsz, 512, n), jnp.float32),
    )(x1, x2, x3, x4, w5)


def kernel(x, W1, W2, W3, W4, W5):
    bsz, c0, n = x.shape
    weights = ((W1, c0), (W2, 64), (W3, 64), (W4, 128))
    hb = bsz // 2
    outs = []
    xt = jnp.transpose(x, (0, 2, 1)).reshape(bsz * n, c0)
    for h in range(2):
        cur, h0 = xt, h * hb
        feats = []
        for w, cin in weights:
            op = max(w.shape[0], 128)
            idx, t, ctr = _tc_layer(cur, w, h0, hb, n, cin, op)
            cur = _sc_layer(t, ctr, idx)
            h0 = 0
            feats.append(cur)
        outs.append(_tc_final(feats, W5, hb, n))
    return jnp.concatenate(outs, axis=0)


# speedup vs baseline: 1.1845x; 1.1845x over previous
import functools

import jax
import jax.numpy as jnp
from jax import lax
from jax.experimental import pallas as pl
from jax.experimental.pallas import tpu as pltpu
from jax.experimental.pallas import tpu_sc as plsc

K = 4


def _tc_layer_body(n, cin, o, op, first, x_ref, w_ref, idx_ref, t_ref,
                   ctr_ref):
    b = pl.program_id(0)
    w = w_ref[...]
    if first:
        xc = x_ref[0]
        g = lax.dot_general(xc, xc, (((0,), (0,)), ((), ())),
                            preferred_element_type=jnp.float32)
        sqc = lax.dot_general(xc * xc, jnp.ones((cin, 1), jnp.float32),
                              (((0,), (0,)), ((), ())),
                              preferred_element_type=jnp.float32)
        tn = lax.dot_general(xc, w[:, :cin], (((0,), (1,)), ((), ())),
                             preferred_element_type=jnp.float32)
        tc = lax.dot_general(xc, w[:, cin:], (((0,), (1,)), ((), ())),
                             preferred_element_type=jnp.float32)
    else:
        xb = x_ref[...][:, :cin]
        sqc = jnp.sum(xb * xb, axis=1, keepdims=True)
        g = lax.dot_general(xb, xb, (((1,), (1,)), ((), ())),
                            preferred_element_type=jnp.float32)
        tn = lax.dot_general(xb, w[:, :cin], (((1,), (1,)), ((), ())),
                             preferred_element_type=jnp.float32)
        tc = lax.dot_general(xb, w[:, cin:], (((1,), (1,)), ((), ())),
                             preferred_element_type=jnp.float32)
    iota0 = lax.broadcasted_iota(jnp.int32, (n, n), 0)
    iotaf = iota0.astype(jnp.float32)
    diag = iota0 == lax.broadcasted_iota(jnp.int32, (n, n), 1)
    d = jnp.where(diag, -jnp.inf, 2.0 * g - sqc)
    idx_ref[0:1, :] = (lax.broadcasted_iota(jnp.int32, (1, n), 1) + b * n)
    for k in range(1, K):
        m = jnp.max(d, axis=0, keepdims=True)
        cand = jnp.where(d == m, iotaf, jnp.float32(n))
        ikf = jnp.min(cand, axis=0, keepdims=True)
        idx_ref[k:k + 1, :] = ikf.astype(jnp.int32) + b * n
        if k < K - 1:
            d = jnp.where(iotaf == ikf, -jnp.inf, d)
    idx_ref[K:, :] = jnp.zeros((8 - K, n), jnp.int32)
    t_ref[:, :o] = tn
    ctr_ref[:, :o] = tc
    if op > o:
        zpad = jnp.zeros((n, op - o), jnp.float32)
        t_ref[:, o:] = zpad
        ctr_ref[:, o:] = zpad


def _tc_layer(xp, w, h0, bsz, n, cin, op):
    o = w.shape[0]
    first = xp.ndim == 3
    if first:
        xspec = pl.BlockSpec((1, cin, n), lambda b, h0=h0: (b + h0, 0, 0))
    else:
        cp = xp.shape[1]
        xspec = pl.BlockSpec((n, cp), lambda b, h0=h0: (b + h0, 0))
    return pl.pallas_call(
        functools.partial(_tc_layer_body, n, cin, o, op, first),
        grid=(bsz,),
        in_specs=[
            xspec,
            pl.BlockSpec(w.shape, lambda b: (0, 0)),
        ],
        out_specs=[
            pl.BlockSpec((8, n), lambda b: (b, 0)),
            pl.BlockSpec((n, op), lambda b: (b, 0)),
            pl.BlockSpec((n, op), lambda b: (b, 0)),
        ],
        out_shape=[
            jax.ShapeDtypeStruct((bsz * 8, n), jnp.int32),
            jax.ShapeDtypeStruct((bsz * n, op), jnp.float32),
            jax.ShapeDtypeStruct((bsz * n, op), jnp.float32),
        ],
    )(xp, w)


def _sc_layer(t, ctr, idx):
    bn, o = t.shape
    n = idx.shape[1]
    nw = 32
    rows_w = bn // nw
    ch = 32
    nch = rows_w // ch
    mesh = plsc.VectorSubcoreMesh(core_axis_name="c", subcore_axis_name="s")

    @functools.partial(
        pl.kernel,
        mesh=mesh,
        out_type=jax.ShapeDtypeStruct((bn, o), jnp.float32),
        scratch_types=[
            pltpu.VMEM((8, rows_w), jnp.int32),
            pltpu.VMEM((ch, o), jnp.float32),
            pltpu.VMEM((ch, o), jnp.float32),
            pltpu.VMEM((ch, o), jnp.float32),
            pltpu.VMEM((ch, o), jnp.float32),
            pltpu.VMEM((ch, o), jnp.float32),
            pltpu.VMEM((ch, o), jnp.float32),
            pltpu.SemaphoreType.DMA,
        ],
    )
    def sc_k(t_hbm, ctr_hbm, idx_hbm, out_hbm, idx_v, r0v, r1v, r2v, r3v,
             ctr_v, out_v, sem):
        wid = lax.axis_index("s") * 2 + lax.axis_index("c")
        base = wid * rows_w
        bi = base // n
        col = base - bi * n
        pltpu.sync_copy(idx_hbm.at[pl.ds(bi * 8, 8), pl.ds(col, rows_w)],
                        idx_v)
        for ci in range(nch):
            p0 = base + ci * ch
            rbufs = (r0v, r1v, r2v, r3v)
            copies = [pltpu.async_copy(t_hbm.at[pl.ds(p0, ch)], r0v, sem)]
            copies += [
                pltpu.async_copy(
                    t_hbm.at[idx_v.at[k, pl.ds(ci * ch, ch)]], rbufs[k], sem)
                for k in range(1, K)]
            pltpu.sync_copy(ctr_hbm.at[pl.ds(p0, ch)], ctr_v)
            for cpy in copies:
                cpy.wait()

            def body(p2, carry):
                for dp in range(2):
                    p = p2 * 2 + dp
                    for j in range(o // 16):
                        s = pl.ds(j * 16, 16)
                        v = jnp.maximum(
                            jnp.maximum(r0v[p, s], r1v[p, s]),
                            jnp.maximum(r2v[p, s], r3v[p, s]))
                        out_v[p, s] = jnp.maximum(v + ctr_v[p, s], 0.0)
                return carry

            lax.fori_loop(0, ch // 2, body, 0)
            pltpu.sync_copy(out_v, out_hbm.at[pl.ds(p0, ch)])

    return sc_k(t, ctr, idx)


def _tc_final_body(x1_ref, x2_ref, x3_ref, x4_ref, w5_ref, out_ref):
    cat = jnp.concatenate(
        [x1_ref[...][:, :64], x2_ref[...][:, :64], x3_ref[...], x4_ref[...]],
        axis=1)
    r = lax.dot_general(w5_ref[...], cat, (((1,), (1,)), ((), ())),
                        preferred_element_type=jnp.float32)
    out_ref[...] = jnp.tanh(r)[None]


def _tc_final(feats, w5, bsz, n):
    x1, x2, x3, x4 = feats
    specs = [pl.BlockSpec((n, f.shape[1]), lambda b: (b, 0)) for f in feats]
    return pl.pallas_call(
        _tc_final_body,
        grid=(bsz,),
        in_specs=specs + [pl.BlockSpec((512, 512), lambda b: (0, 0))],
        out_specs=pl.BlockSpec((1, 512, n), lambda b: (b, 0, 0)),
        out_shape=jax.ShapeDtypeStruct((bsz, 512, n), jnp.float32),
    )(x1, x2, x3, x4, w5)


def kernel(x, W1, W2, W3, W4, W5):
    bsz, c0, n = x.shape
    weights = ((W1, c0), (W2, 64), (W3, 64), (W4, 128))
    hb = bsz // 2
    outs = []
    xt = jnp.transpose(x, (0, 2, 1)).reshape(bsz * n, c0)
    for h in range(2):
        cur, h0 = xt, h * hb
        feats = []
        for w, cin in weights:
            op = max(w.shape[0], 128)
            idx, t, ctr = _tc_layer(cur, w, h0, hb, n, cin, op)
            cur = _sc_layer(t, ctr, idx)
            h0 = 0
            feats.append(cur)
        outs.append(_tc_final(feats, W5, hb, n))
    return jnp.concatenate(outs, axis=0)


# speedup vs baseline: 1.3558x; 1.1446x over previous
import functools

import jax
import jax.numpy as jnp
from jax import lax
from jax.experimental import pallas as pl
from jax.experimental.pallas import tpu as pltpu
from jax.experimental.pallas import tpu_sc as plsc

K = 4


def _tc_layer_body(n, cin, o, op, first, x_ref, w_ref, idx_ref, t_ref,
                   ctr_ref):
    b = pl.program_id(0)
    w = w_ref[...]
    if first:
        xc = x_ref[0]
        g = lax.dot_general(xc, xc, (((0,), (0,)), ((), ())),
                            preferred_element_type=jnp.float32)
        sqc = lax.dot_general(xc * xc, jnp.ones((cin, 1), jnp.float32),
                              (((0,), (0,)), ((), ())),
                              preferred_element_type=jnp.float32)
        tn = lax.dot_general(xc, w[:, :cin], (((0,), (1,)), ((), ())),
                             preferred_element_type=jnp.float32)
        tc = lax.dot_general(xc, w[:, cin:], (((0,), (1,)), ((), ())),
                             preferred_element_type=jnp.float32)
    else:
        xb = x_ref[...][:, :cin]
        sqc = jnp.sum(xb * xb, axis=1, keepdims=True)
        g = lax.dot_general(xb, xb, (((1,), (1,)), ((), ())),
                            preferred_element_type=jnp.float32)
        tn = lax.dot_general(xb, w[:, :cin], (((1,), (1,)), ((), ())),
                             preferred_element_type=jnp.float32)
        tc = lax.dot_general(xb, w[:, cin:], (((1,), (1,)), ((), ())),
                             preferred_element_type=jnp.float32)
    iota0 = lax.broadcasted_iota(jnp.int32, (n, n), 0)
    iotaf = iota0.astype(jnp.float32)
    diag = iota0 == lax.broadcasted_iota(jnp.int32, (n, n), 1)
    d = jnp.where(diag, -jnp.inf, 2.0 * g - sqc)
    idx_ref[0:1, :] = (lax.broadcasted_iota(jnp.int32, (1, n), 1) + b * n)
    for k in range(1, K):
        m = jnp.max(d, axis=0, keepdims=True)
        cand = jnp.where(d == m, iotaf, jnp.float32(n))
        ikf = jnp.min(cand, axis=0, keepdims=True)
        idx_ref[k:k + 1, :] = ikf.astype(jnp.int32) + b * n
        if k < K - 1:
            d = jnp.where(iotaf == ikf, -jnp.inf, d)
    idx_ref[K:, :] = jnp.zeros((8 - K, n), jnp.int32)
    t_ref[:, :o] = tn
    ctr_ref[:, :o] = tc
    if op > o:
        zpad = jnp.zeros((n, op - o), jnp.float32)
        t_ref[:, o:] = zpad
        ctr_ref[:, o:] = zpad


def _tc_layer(xp, w, h0, bsz, n, cin, op):
    o = w.shape[0]
    first = xp.ndim == 3
    if first:
        xspec = pl.BlockSpec((1, cin, n), lambda b, h0=h0: (b + h0, 0, 0))
    else:
        cp = xp.shape[1]
        xspec = pl.BlockSpec((n, cp), lambda b, h0=h0: (b + h0, 0))
    return pl.pallas_call(
        functools.partial(_tc_layer_body, n, cin, o, op, first),
        grid=(bsz,),
        in_specs=[
            xspec,
            pl.BlockSpec(w.shape, lambda b: (0, 0)),
        ],
        out_specs=[
            pl.BlockSpec((8, n), lambda b: (b, 0)),
            pl.BlockSpec((n, op), lambda b: (b, 0)),
            pl.BlockSpec((n, op), lambda b: (b, 0)),
        ],
        out_shape=[
            jax.ShapeDtypeStruct((bsz * 8, n), jnp.int32),
            jax.ShapeDtypeStruct((bsz * n, op), jnp.float32),
            jax.ShapeDtypeStruct((bsz * n, op), jnp.float32),
        ],
    )(xp, w)


def _sc_layer(t, ctr, idx):
    bn, o = t.shape
    n = idx.shape[1]
    nw = 32
    rows_w = bn // nw
    ch = 32
    nch = rows_w // ch
    mesh = plsc.VectorSubcoreMesh(core_axis_name="c", subcore_axis_name="s")

    @functools.partial(
        pl.kernel,
        mesh=mesh,
        out_type=jax.ShapeDtypeStruct((bn, o), jnp.float32),
        scratch_types=[
            pltpu.VMEM((8, rows_w), jnp.int32),
            pltpu.VMEM((ch, o), jnp.float32),
            pltpu.VMEM((ch, o), jnp.float32),
            pltpu.VMEM((ch, o), jnp.float32),
            pltpu.VMEM((ch, o), jnp.float32),
            pltpu.VMEM((ch, o), jnp.float32),
            pltpu.VMEM((ch, o), jnp.float32),
            pltpu.SemaphoreType.DMA,
        ],
    )
    def sc_k(t_hbm, ctr_hbm, idx_hbm, out_hbm, idx_v, r0v, r1v, r2v, r3v,
             ctr_v, out_v, sem):
        wid = lax.axis_index("s") * 2 + lax.axis_index("c")
        base = wid * rows_w
        bi = base // n
        col = base - bi * n
        pltpu.sync_copy(idx_hbm.at[pl.ds(bi * 8, 8), pl.ds(col, rows_w)],
                        idx_v)
        for ci in range(nch):
            p0 = base + ci * ch
            rbufs = (r0v, r1v, r2v, r3v)
            copies = [pltpu.async_copy(t_hbm.at[pl.ds(p0, ch)], r0v, sem)]
            copies += [
                pltpu.async_copy(
                    t_hbm.at[idx_v.at[k, pl.ds(ci * ch, ch)]], rbufs[k], sem)
                for k in range(1, K)]
            pltpu.sync_copy(ctr_hbm.at[pl.ds(p0, ch)], ctr_v)
            for cpy in copies:
                cpy.wait()

            def body(p, carry):
                for j in range(o // 16):
                    s = pl.ds(j * 16, 16)
                    v = jnp.maximum(
                        jnp.maximum(r0v[p, s], r1v[p, s]),
                        jnp.maximum(r2v[p, s], r3v[p, s]))
                    out_v[p, s] = jnp.maximum(v + ctr_v[p, s], 0.0)
                return carry

            lax.fori_loop(0, ch, body, 0)
            pltpu.sync_copy(out_v, out_hbm.at[pl.ds(p0, ch)])

    return sc_k(t, ctr, idx)


def _tc_final_body(*refs):
    x1_ref, x2_ref, x3_ref, x4_ref, w5_ref = refs[:5]
    out_ref = refs[-1]
    cat = jnp.concatenate(
        [x1_ref[...][:, :64], x2_ref[...][:, :64], x3_ref[...], x4_ref[...]],
        axis=1)
    r = lax.dot_general(w5_ref[...], cat, (((1,), (1,)), ((), ())),
                        preferred_element_type=jnp.float32)
    out_ref[...] = jnp.tanh(r)[None]


def _tc_final(feats, w5, bsz, n, base, bsz_total, prev=None):
    x1, x2, x3, x4 = feats
    specs = [pl.BlockSpec((n, f.shape[1]), lambda b: (b, 0)) for f in feats]
    specs.append(pl.BlockSpec((512, 512), lambda b: (0, 0)))
    args = [x1, x2, x3, x4, w5]
    aliases = {}
    if prev is not None:
        specs.append(pl.BlockSpec((1, 8, 128), lambda b: (0, 0, 0)))
        args.append(prev)
        aliases = {5: 0}
    return pl.pallas_call(
        _tc_final_body,
        grid=(bsz,),
        in_specs=specs,
        out_specs=pl.BlockSpec((1, 512, n), lambda b, base=base: (b + base,
                                                                  0, 0)),
        out_shape=jax.ShapeDtypeStruct((bsz_total, 512, n), jnp.float32),
        input_output_aliases=aliases,
    )(*args)


def kernel(x, W1, W2, W3, W4, W5):
    bsz, c0, n = x.shape
    weights = ((W1, c0), (W2, 64), (W3, 64), (W4, 128))
    hb = bsz // 2
    outs = []
    xt = jnp.transpose(x, (0, 2, 1)).reshape(bsz * n, c0)
    for h in range(2):
        cur, h0 = xt, h * hb
        feats = []
        for w, cin in weights:
            op = max(w.shape[0], 128)
            idx, t, ctr = _tc_layer(cur, w, h0, hb, n, cin, op)
            cur = _sc_layer(t, ctr, idx)
            h0 = 0
            feats.append(cur)
        outs.append(_tc_final(feats, W5, hb, n, h * hb, bsz,
                              prev=outs[0] if h else None))
    return outs[1]
